# baseline (speedup 1.0000x reference)
import functools

import jax
import jax.numpy as jnp
from jax.experimental import pallas as pl

N = 1024
DIM = 512
TILE = 128


def _fused_kernel(adj_ref, x_ref, w1a_ref, w1b_ref, b1_ref, w2_ref, b2_ref,
                  wn1_ref, bn1_ref, wn2_ref, bn2_ref, out_ref):
    j = pl.program_id(0)
    f32 = jnp.float32

    adj_f = (adj_ref[...] > 0).astype(f32)
    counts = jnp.sum(adj_f, axis=0)
    pa = jax.lax.dot_general(
        adj_f, x_ref[...],
        dimension_numbers=(((0,), (0,)), ((), ())),
        preferred_element_type=f32)
    inv = jnp.where(counts > 0, 1.0 / jnp.where(counts > 0, counts, 1.0), 0.0)
    pa = pa * inv[:, None]

    xt = x_ref[pl.ds(j * TILE, TILE), :]

    def mm(a, w):
        return jax.lax.dot_general(
            a, w, dimension_numbers=(((1,), (1,)), ((), ())),
            preferred_element_type=f32)

    pre1 = mm(xt, w1a_ref[...]) + mm(pa, w1b_ref[...]) + b1_ref[...]
    h1 = jnp.maximum(pre1, 0.0)
    h = mm(h1, w2_ref[...]) + b2_ref[...]

    n1 = jnp.maximum(mm(xt, wn1_ref[...]) + bn1_ref[...], 0.0)
    noise = mm(n1, wn2_ref[...]) + bn2_ref[...]

    out_ref[...] = jnp.where(counts[:, None] > 0, h + noise, xt)


@functools.partial(jax.jit, static_argnames=())
def kernel(variable_embeddings, adjacency, W1, b1, W2, b2, Wn1, bn1, Wn2, bn2):
    x = variable_embeddings
    W1a = W1[:, :DIM]
    W1b = W1[:, DIM:]
    grid = (N // TILE,)
    full = lambda r, c: pl.BlockSpec((r, c), lambda j: (0, 0))
    out = pl.pallas_call(
        _fused_kernel,
        grid=grid,
        in_specs=[
            pl.BlockSpec((N, TILE), lambda j: (0, j)),
            full(N, DIM),
            full(DIM, DIM),
            full(DIM, DIM),
            full(1, DIM),
            full(DIM, DIM),
            full(1, DIM),
            full(DIM // 2, DIM),
            full(1, DIM // 2),
            full(DIM, DIM // 2),
            full(1, DIM),
        ],
        out_specs=pl.BlockSpec((TILE, DIM), lambda j: (j, 0)),
        out_shape=jax.ShapeDtypeStruct((N, DIM), jnp.float32),
    )(adjacency, x, W1a, W1b, b1.reshape(1, DIM), W2, b2.reshape(1, DIM),
      Wn1, bn1.reshape(1, DIM // 2), Wn2, bn2.reshape(1, DIM))
    return out


# speedup vs baseline: 1.4802x; 1.4802x over previous
import functools

import jax
import jax.numpy as jnp
from jax.experimental import pallas as pl

N = 1024
DIM = 512
TILE = 512


def _fused_kernel(adj_ref, x_ref, w1a_ref, w1b_ref, b1_ref, w2_ref, b2_ref,
                  wn1_ref, bn1_ref, wn2_ref, bn2_ref, out_ref):
    j = pl.program_id(0)
    f32 = jnp.float32

    adj_f = (adj_ref[...] > 0).astype(f32)
    counts = jnp.sum(adj_f, axis=0)
    pa = jax.lax.dot_general(
        adj_f, x_ref[...],
        dimension_numbers=(((0,), (0,)), ((), ())),
        preferred_element_type=f32)
    inv = jnp.where(counts > 0, 1.0 / jnp.where(counts > 0, counts, 1.0), 0.0)
    pa = pa * inv[:, None]

    xt = x_ref[pl.ds(j * TILE, TILE), :]

    def mm(a, w):
        return jax.lax.dot_general(
            a, w, dimension_numbers=(((1,), (1,)), ((), ())),
            preferred_element_type=f32)

    pre1 = mm(xt, w1a_ref[...]) + mm(pa, w1b_ref[...]) + b1_ref[...]
    h1 = jnp.maximum(pre1, 0.0)
    h = mm(h1, w2_ref[...]) + b2_ref[...]

    n1 = jnp.maximum(mm(xt, wn1_ref[...]) + bn1_ref[...], 0.0)
    noise = mm(n1, wn2_ref[...]) + bn2_ref[...]

    out_ref[...] = jnp.where(counts[:, None] > 0, h + noise, xt)


@functools.partial(jax.jit, static_argnames=())
def kernel(variable_embeddings, adjacency, W1, b1, W2, b2, Wn1, bn1, Wn2, bn2):
    x = variable_embeddings
    W1a = W1[:, :DIM]
    W1b = W1[:, DIM:]
    grid = (N // TILE,)
    full = lambda r, c: pl.BlockSpec((r, c), lambda j: (0, 0))
    out = pl.pallas_call(
        _fused_kernel,
        grid=grid,
        in_specs=[
            pl.BlockSpec((N, TILE), lambda j: (0, j)),
            full(N, DIM),
            full(DIM, DIM),
            full(DIM, DIM),
            full(1, DIM),
            full(DIM, DIM),
            full(1, DIM),
            full(DIM // 2, DIM),
            full(1, DIM // 2),
            full(DIM, DIM // 2),
            full(1, DIM),
        ],
        out_specs=pl.BlockSpec((TILE, DIM), lambda j: (j, 0)),
        out_shape=jax.ShapeDtypeStruct((N, DIM), jnp.float32),
    )(adjacency, x, W1a, W1b, b1.reshape(1, DIM), W2, b2.reshape(1, DIM),
      Wn1, bn1.reshape(1, DIM // 2), Wn2, bn2.reshape(1, DIM))
    return out
